# baseline (speedup 1.0000x reference)
import functools
import math

import jax
import jax.numpy as jnp
from jax import lax
from jax.experimental import pallas as pl
from jax.experimental.pallas import tpu as pltpu
from jax.experimental.pallas import tpu_sc as plsc

_N = 2048
_D = 768
_H = 3072
_E = 8
_T = 128
_NPAD = 3072
_NT = _NPAD // _T
_NW = 32


def _route_body(x_ref, wg_ref, bg_ref, dest_ref, poffs_ref, ntiles_ref):
    l = jnp.dot(x_ref[...], wg_ref[...], preferred_element_type=jnp.float32)
    l = l + bg_ref[...]
    idx = lax.broadcasted_iota(jnp.int32, l.shape, 1)
    m1 = jnp.max(l, axis=1, keepdims=True)
    i1 = jnp.min(jnp.where(l == m1, idx, _E), axis=1, keepdims=True)
    l2 = jnp.where(idx == i1, -jnp.inf, l)
    m2 = jnp.max(l2, axis=1, keepdims=True)
    i2 = jnp.min(jnp.where(l2 == m2, idx, _E), axis=1, keepdims=True)
    e_win = jnp.maximum(i1, i2)

    onehot = (idx == e_win).astype(jnp.float32)
    ri = lax.broadcasted_iota(jnp.int32, (_N, _N), 0)
    ci = lax.broadcasted_iota(jnp.int32, (_N, _N), 1)
    tri = (ci < ri).astype(jnp.bfloat16)
    rank = jnp.dot(tri, onehot.astype(jnp.bfloat16),
                   preferred_element_type=jnp.float32)
    counts = jnp.sum(onehot, axis=0, keepdims=True).astype(jnp.int32)
    pcounts = ((counts + (_T - 1)) // _T) * _T
    er = lax.broadcasted_iota(jnp.int32, (_E, _E), 0)
    ec = lax.broadcasted_iota(jnp.int32, (_E, _E), 1)
    incl = (er <= ec).astype(jnp.float32)
    pends = jnp.dot(pcounts.astype(jnp.float32), incl,
                    preferred_element_type=jnp.float32).astype(jnp.int32)
    poffs = pends - pcounts
    rank_t = jnp.sum(rank * onehot, axis=1, keepdims=True)
    poff_t = jnp.sum(poffs.astype(jnp.float32) * onehot, axis=1,
                     keepdims=True)
    dest_ref[...] = (poff_t + rank_t).astype(jnp.int32)
    poffs_ref[...] = poffs.reshape(_E, 1)
    ntiles_ref[...] = (pcounts // _T).reshape(_E, 1)


def _route(xf, Wg, bg2):
    return pl.pallas_call(
        _route_body,
        out_shape=(
            jax.ShapeDtypeStruct((_N, 1), jnp.int32),
            jax.ShapeDtypeStruct((_E, 1), jnp.int32),
            jax.ShapeDtypeStruct((_E, 1), jnp.int32),
        ),
    )(xf, Wg, bg2)


_KH = 4
_HC = _H // _KH


def _ffn_body(poffs_ref, ntiles_ref, x_ref, w1_ref, b1_ref, w2_ref, b2_ref,
              o_ref):
    e = pl.program_id(0)
    k = pl.program_id(1)
    off = pl.multiple_of(poffs_ref[e], _T)
    nt = ntiles_ref[e]

    def tile_body(i, _):
        sl = pl.ds(pl.multiple_of(off + i * _T, _T), _T)
        xt = x_ref[sl, :]
        h = jnp.dot(xt, w1_ref[0], preferred_element_type=jnp.float32)
        h = h + b1_ref[0]
        h = 0.5 * h * (1.0 + lax.erf(h * (1.0 / math.sqrt(2.0))))
        yk = 2.0 * jnp.dot(h, w2_ref[0], preferred_element_type=jnp.float32)

        @pl.when(k == 0)
        def _():
            o_ref[sl, :] = yk + 2.0 * b2_ref[0]

        @pl.when(k > 0)
        def _():
            o_ref[sl, :] = o_ref[sl, :] + yk

        return 0

    lax.fori_loop(0, nt, tile_body, 0)


def _ffn(poffs, ntiles, x_pad, W1, b1, W2, b2):
    grid_spec = pltpu.PrefetchScalarGridSpec(
        num_scalar_prefetch=2,
        grid=(_E, _KH),
        in_specs=[
            pl.BlockSpec((_NPAD, _D), lambda e, k, po, nt: (0, 0)),
            pl.BlockSpec((1, _D, _HC), lambda e, k, po, nt: (e, 0, k)),
            pl.BlockSpec((1, 1, _HC), lambda e, k, po, nt: (e, 0, k)),
            pl.BlockSpec((1, _HC, _D), lambda e, k, po, nt: (e, k, 0)),
            pl.BlockSpec((1, 1, _D), lambda e, k, po, nt: (e, 0, 0)),
        ],
        out_specs=pl.BlockSpec((_NPAD, _D), lambda e, k, po, nt: (0, 0)),
    )
    return pl.pallas_call(
        _ffn_body,
        grid_spec=grid_spec,
        out_shape=jax.ShapeDtypeStruct((_NPAD, _D), jnp.float32),
    )(poffs, ntiles, x_pad, W1, b1.reshape(_E, 1, _H), W2,
      b2.reshape(_E, 1, _D))


_BW = _N // _NW
_mesh = plsc.VectorSubcoreMesh(core_axis_name="c", subcore_axis_name="s")


@functools.partial(
    pl.kernel,
    mesh=_mesh,
    out_type=jax.ShapeDtypeStruct((_NPAD, _D), jnp.float32),
    scratch_types=[
        pltpu.VMEM((_BW,), jnp.int32),
        pltpu.VMEM((_BW, _D), jnp.float32),
        pltpu.SemaphoreType.DMA,
    ],
)
def _sc_dispatch(x_hbm, dest_hbm, out_hbm, idx_v, rows_v, sem):
    wid = lax.axis_index("s") * 2 + lax.axis_index("c")
    pltpu.sync_copy(dest_hbm.at[wid], idx_v)
    pltpu.sync_copy(x_hbm.at[pl.ds(wid * _BW, _BW)], rows_v)
    pltpu.async_copy(rows_v, out_hbm.at[idx_v], sem).wait()


@functools.partial(
    pl.kernel,
    mesh=_mesh,
    out_type=jax.ShapeDtypeStruct((_N, _D), jnp.float32),
    scratch_types=[
        pltpu.VMEM((_BW,), jnp.int32),
        pltpu.VMEM((_BW, _D), jnp.float32),
        pltpu.SemaphoreType.DMA,
    ],
)
def _sc_combine(y_hbm, dest_hbm, out_hbm, idx_v, rows_v, sem):
    wid = lax.axis_index("s") * 2 + lax.axis_index("c")
    pltpu.sync_copy(dest_hbm.at[wid], idx_v)
    pltpu.async_copy(y_hbm.at[idx_v], rows_v, sem).wait()
    pltpu.sync_copy(rows_v, out_hbm.at[pl.ds(wid * _BW, _BW)])


def kernel(x, Wg, bg, W1, b1, W2, b2):
    xf = x.reshape(_N, _D)
    dest, poffs, ntiles = _route(xf, Wg, bg.reshape(1, _E))
    dest2 = dest.reshape(_NW, _BW)
    x_pad = _sc_dispatch(xf, dest2)
    y_pad = _ffn(poffs[:, 0], ntiles[:, 0], x_pad, W1, b1, W2, b2)
    out = _sc_combine(y_pad, dest2)
    return out.reshape(x.shape)


# speedup vs baseline: 1.1296x; 1.1296x over previous
import functools
import math

import jax
import jax.numpy as jnp
from jax import lax
from jax.experimental import pallas as pl
from jax.experimental.pallas import tpu as pltpu
from jax.experimental.pallas import tpu_sc as plsc

_N = 2048
_D = 768
_H = 3072
_E = 8
_T = 128
_NPAD = 3072
_NT = _NPAD // _T
_NW = 32


def _route_body(x_ref, wg_ref, bg_ref, dest_ref, poffs_ref, ntiles_ref):
    l = jnp.dot(x_ref[...], wg_ref[...], preferred_element_type=jnp.float32)
    l = l + bg_ref[...]
    idx = lax.broadcasted_iota(jnp.int32, l.shape, 1)
    m1 = jnp.max(l, axis=1, keepdims=True)
    i1 = jnp.min(jnp.where(l == m1, idx, _E), axis=1, keepdims=True)
    l2 = jnp.where(idx == i1, -jnp.inf, l)
    m2 = jnp.max(l2, axis=1, keepdims=True)
    i2 = jnp.min(jnp.where(l2 == m2, idx, _E), axis=1, keepdims=True)
    e_win = jnp.maximum(i1, i2)

    onehot = (idx == e_win).astype(jnp.float32)
    ri = lax.broadcasted_iota(jnp.int32, (_N, _N), 0)
    ci = lax.broadcasted_iota(jnp.int32, (_N, _N), 1)
    tri = (ci < ri).astype(jnp.bfloat16)
    rank = jnp.dot(tri, onehot.astype(jnp.bfloat16),
                   preferred_element_type=jnp.float32)
    counts = jnp.sum(onehot, axis=0, keepdims=True).astype(jnp.int32)
    pcounts = ((counts + (_T - 1)) // _T) * _T
    er = lax.broadcasted_iota(jnp.int32, (_E, _E), 0)
    ec = lax.broadcasted_iota(jnp.int32, (_E, _E), 1)
    incl = (er <= ec).astype(jnp.float32)
    pends = jnp.dot(pcounts.astype(jnp.float32), incl,
                    preferred_element_type=jnp.float32).astype(jnp.int32)
    poffs = pends - pcounts
    rank_t = jnp.sum(rank * onehot, axis=1, keepdims=True)
    poff_t = jnp.sum(poffs.astype(jnp.float32) * onehot, axis=1,
                     keepdims=True)
    dest_ref[...] = (poff_t + rank_t).astype(jnp.int32)
    poffs_ref[...] = poffs.reshape(_E, 1)
    ntiles_ref[...] = (pcounts // _T).reshape(_E, 1)


def _route(xf, Wg, bg2):
    return pl.pallas_call(
        _route_body,
        out_shape=(
            jax.ShapeDtypeStruct((_N, 1), jnp.int32),
            jax.ShapeDtypeStruct((_E, 1), jnp.int32),
            jax.ShapeDtypeStruct((_E, 1), jnp.int32),
        ),
    )(xf, Wg, bg2)


_KH = 2
_HC = _H // _KH


def _ffn_body(poffs_ref, ntiles_ref, x_ref, w1_ref, b1_ref, w2_ref, b2_ref,
              o_ref):
    e = pl.program_id(0)
    k = pl.program_id(1)
    off = pl.multiple_of(poffs_ref[e], _T)
    nt = ntiles_ref[e]

    def tile_body(i, _):
        sl = pl.ds(pl.multiple_of(off + i * _T, _T), _T)
        xt = x_ref[sl, :]
        h = jnp.dot(xt, w1_ref[0], preferred_element_type=jnp.float32)
        h = h + b1_ref[0]
        h = 0.5 * h * (1.0 + lax.erf(h * (1.0 / math.sqrt(2.0))))
        yk = 2.0 * jnp.dot(h, w2_ref[0], preferred_element_type=jnp.float32)

        @pl.when(k == 0)
        def _():
            o_ref[sl, :] = yk + 2.0 * b2_ref[0]

        @pl.when(k > 0)
        def _():
            o_ref[sl, :] = o_ref[sl, :] + yk

        return 0

    lax.fori_loop(0, nt, tile_body, 0)


def _ffn(poffs, ntiles, x_pad, W1, b1, W2, b2):
    grid_spec = pltpu.PrefetchScalarGridSpec(
        num_scalar_prefetch=2,
        grid=(_E, _KH),
        in_specs=[
            pl.BlockSpec((_NPAD, _D), lambda e, k, po, nt: (0, 0)),
            pl.BlockSpec((1, _D, _HC), lambda e, k, po, nt: (e, 0, k)),
            pl.BlockSpec((1, 1, _HC), lambda e, k, po, nt: (e, 0, k)),
            pl.BlockSpec((1, _HC, _D), lambda e, k, po, nt: (e, k, 0)),
            pl.BlockSpec((1, 1, _D), lambda e, k, po, nt: (e, 0, 0)),
        ],
        out_specs=pl.BlockSpec((_NPAD, _D), lambda e, k, po, nt: (0, 0)),
    )
    return pl.pallas_call(
        _ffn_body,
        grid_spec=grid_spec,
        out_shape=jax.ShapeDtypeStruct((_NPAD, _D), jnp.float32),
    )(poffs, ntiles, x_pad, W1, b1.reshape(_E, 1, _H), W2,
      b2.reshape(_E, 1, _D))


_BW = _N // _NW
_mesh = plsc.VectorSubcoreMesh(core_axis_name="c", subcore_axis_name="s")


@functools.partial(
    pl.kernel,
    mesh=_mesh,
    out_type=jax.ShapeDtypeStruct((_NPAD, _D), jnp.float32),
    scratch_types=[
        pltpu.VMEM((_BW,), jnp.int32),
        pltpu.VMEM((_BW, _D), jnp.float32),
        pltpu.SemaphoreType.DMA,
    ],
)
def _sc_dispatch(x_hbm, dest_hbm, out_hbm, idx_v, rows_v, sem):
    wid = lax.axis_index("s") * 2 + lax.axis_index("c")
    pltpu.sync_copy(dest_hbm.at[wid], idx_v)
    pltpu.sync_copy(x_hbm.at[pl.ds(wid * _BW, _BW)], rows_v)
    pltpu.async_copy(rows_v, out_hbm.at[idx_v], sem).wait()


@functools.partial(
    pl.kernel,
    mesh=_mesh,
    out_type=jax.ShapeDtypeStruct((_N, _D), jnp.float32),
    scratch_types=[
        pltpu.VMEM((_BW,), jnp.int32),
        pltpu.VMEM((_BW, _D), jnp.float32),
        pltpu.SemaphoreType.DMA,
    ],
)
def _sc_combine(y_hbm, dest_hbm, out_hbm, idx_v, rows_v, sem):
    wid = lax.axis_index("s") * 2 + lax.axis_index("c")
    pltpu.sync_copy(dest_hbm.at[wid], idx_v)
    pltpu.async_copy(y_hbm.at[idx_v], rows_v, sem).wait()
    pltpu.sync_copy(rows_v, out_hbm.at[pl.ds(wid * _BW, _BW)])


def kernel(x, Wg, bg, W1, b1, W2, b2):
    xf = x.reshape(_N, _D)
    dest, poffs, ntiles = _route(xf, Wg, bg.reshape(1, _E))
    dest2 = dest.reshape(_NW, _BW)
    x_pad = _sc_dispatch(xf, dest2)
    y_pad = _ffn(poffs[:, 0], ntiles[:, 0], x_pad, W1, b1, W2, b2)
    out = _sc_combine(y_pad, dest2)
    return out.reshape(x.shape)


# speedup vs baseline: 1.1314x; 1.0017x over previous
import functools
import math

import jax
import jax.numpy as jnp
from jax import lax
from jax.experimental import pallas as pl
from jax.experimental.pallas import tpu as pltpu
from jax.experimental.pallas import tpu_sc as plsc

_N = 2048
_D = 768
_H = 3072
_E = 8
_T = 128
_NPAD = 3072
_NT = _NPAD // _T
_NW = 32


def _route_body(x_ref, wg_ref, bg_ref, dest_ref, poffs_ref, ntiles_ref):
    l = jnp.dot(x_ref[0], wg_ref[...], preferred_element_type=jnp.float32)
    l = l + bg_ref[...]
    idx = lax.broadcasted_iota(jnp.int32, l.shape, 1)
    m1 = jnp.max(l, axis=1, keepdims=True)
    i1 = jnp.min(jnp.where(l == m1, idx, _E), axis=1, keepdims=True)
    l2 = jnp.where(idx == i1, -jnp.inf, l)
    m2 = jnp.max(l2, axis=1, keepdims=True)
    i2 = jnp.min(jnp.where(l2 == m2, idx, _E), axis=1, keepdims=True)
    e_win = jnp.maximum(i1, i2)

    onehot = (idx == e_win).astype(jnp.float32)
    ri = lax.broadcasted_iota(jnp.int32, (_N, _N), 0)
    ci = lax.broadcasted_iota(jnp.int32, (_N, _N), 1)
    tri = (ci < ri).astype(jnp.bfloat16)
    rank = jnp.dot(tri, onehot.astype(jnp.bfloat16),
                   preferred_element_type=jnp.float32)
    counts = jnp.sum(onehot, axis=0, keepdims=True).astype(jnp.int32)
    pcounts = ((counts + (_T - 1)) // _T) * _T
    er = lax.broadcasted_iota(jnp.int32, (_E, _E), 0)
    ec = lax.broadcasted_iota(jnp.int32, (_E, _E), 1)
    incl = (er <= ec).astype(jnp.float32)
    pends = jnp.dot(pcounts.astype(jnp.float32), incl,
                    preferred_element_type=jnp.float32).astype(jnp.int32)
    poffs = pends - pcounts
    rank_t = jnp.sum(rank * onehot, axis=1, keepdims=True)
    poff_t = jnp.sum(poffs.astype(jnp.float32) * onehot, axis=1,
                     keepdims=True)
    dest_ref[...] = (poff_t + rank_t).astype(jnp.int32)
    poffs_ref[...] = poffs.reshape(_E, 1)
    ntiles_ref[...] = (pcounts // _T).reshape(_E, 1)


def _route(xf, Wg, bg2):
    return pl.pallas_call(
        _route_body,
        out_shape=(
            jax.ShapeDtypeStruct((_N, 1), jnp.int32),
            jax.ShapeDtypeStruct((_E, 1), jnp.int32),
            jax.ShapeDtypeStruct((_E, 1), jnp.int32),
        ),
    )(xf, Wg, bg2)


_KH = 2
_HC = _H // _KH


def _ffn_body(poffs_ref, ntiles_ref, x_ref, w1_ref, b1_ref, w2_ref, b2_ref,
              o_ref):
    e = pl.program_id(0)
    k = pl.program_id(1)
    off = pl.multiple_of(poffs_ref[e], _T)
    nt = ntiles_ref[e]

    def tile_body(i, _):
        sl = pl.ds(pl.multiple_of(off + i * _T, _T), _T)
        xt = x_ref[sl, :]
        h = jnp.dot(xt, w1_ref[0], preferred_element_type=jnp.float32)
        h = h + b1_ref[0]
        h = 0.5 * h * (1.0 + lax.erf(h * (1.0 / math.sqrt(2.0))))
        yk = 2.0 * jnp.dot(h, w2_ref[0], preferred_element_type=jnp.float32)

        @pl.when(k == 0)
        def _():
            o_ref[sl, :] = yk + 2.0 * b2_ref[0]

        @pl.when(k > 0)
        def _():
            o_ref[sl, :] = o_ref[sl, :] + yk

        return 0

    lax.fori_loop(0, nt, tile_body, 0)


def _ffn(poffs, ntiles, x_pad, W1, b1, W2, b2):
    grid_spec = pltpu.PrefetchScalarGridSpec(
        num_scalar_prefetch=2,
        grid=(_E, _KH),
        in_specs=[
            pl.BlockSpec((_NPAD, _D), lambda e, k, po, nt: (0, 0)),
            pl.BlockSpec((1, _D, _HC), lambda e, k, po, nt: (e, 0, k)),
            pl.BlockSpec((1, 1, _HC), lambda e, k, po, nt: (e, 0, k)),
            pl.BlockSpec((1, _HC, _D), lambda e, k, po, nt: (e, k, 0)),
            pl.BlockSpec((1, 1, _D), lambda e, k, po, nt: (e, 0, 0)),
        ],
        out_specs=pl.BlockSpec((_NPAD, _D), lambda e, k, po, nt: (0, 0)),
    )
    return pl.pallas_call(
        _ffn_body,
        grid_spec=grid_spec,
        out_shape=jax.ShapeDtypeStruct((_NPAD, _D), jnp.float32),
    )(poffs, ntiles, x_pad, W1, b1.reshape(_E, 1, _H), W2,
      b2.reshape(_E, 1, _D))


_BW = _N // _NW
_mesh = plsc.VectorSubcoreMesh(core_axis_name="c", subcore_axis_name="s")


@functools.partial(
    pl.kernel,
    mesh=_mesh,
    out_type=jax.ShapeDtypeStruct((_NPAD, _D), jnp.float32),
    scratch_types=[
        pltpu.VMEM((_BW,), jnp.int32),
        pltpu.VMEM((_BW, _D), jnp.float32),
        pltpu.SemaphoreType.DMA,
    ],
)
def _sc_dispatch(x_hbm, dest_hbm, out_hbm, idx_v, rows_v, sem):
    wid = lax.axis_index("s") * 2 + lax.axis_index("c")
    pltpu.sync_copy(dest_hbm.at[wid], idx_v)
    pltpu.sync_copy(x_hbm.at[0, pl.ds(wid * _BW, _BW)], rows_v)
    pltpu.async_copy(rows_v, out_hbm.at[idx_v], sem).wait()


@functools.partial(
    pl.kernel,
    mesh=_mesh,
    out_type=jax.ShapeDtypeStruct((1, _N, _D), jnp.float32),
    scratch_types=[
        pltpu.VMEM((_BW,), jnp.int32),
        pltpu.VMEM((_BW, _D), jnp.float32),
        pltpu.SemaphoreType.DMA,
    ],
)
def _sc_combine(y_hbm, dest_hbm, out_hbm, idx_v, rows_v, sem):
    wid = lax.axis_index("s") * 2 + lax.axis_index("c")
    pltpu.sync_copy(dest_hbm.at[wid], idx_v)
    pltpu.async_copy(y_hbm.at[idx_v], rows_v, sem).wait()
    pltpu.sync_copy(rows_v, out_hbm.at[0, pl.ds(wid * _BW, _BW)])


def kernel(x, Wg, bg, W1, b1, W2, b2):
    dest, poffs, ntiles = _route(x, Wg, bg.reshape(1, _E))
    dest2 = dest.reshape(_NW, _BW)
    x_pad = _sc_dispatch(x, dest2)
    y_pad = _ffn(poffs[:, 0], ntiles[:, 0], x_pad, W1, b1, W2, b2)
    return _sc_combine(y_pad, dest2)
